# Initial kernel scaffold; baseline (speedup 1.0000x reference)
import jax
import jax.numpy as jnp
from jax.experimental import pallas as pl


def kernel(edge_index, user_emb, item_emb):
    raise NotImplementedError("write your pallas kernel here")


# speedup vs baseline: 7.1192x; 7.1192x over previous
import functools

import jax
import jax.numpy as jnp
from jax import lax
from jax.experimental import pallas as pl
from jax.experimental.pallas import tpu as pltpu
from jax.experimental.pallas import tpu_sc as plsc

NU = 25000
NP = 50176
HALF = 25088
PT = 1568
TRASH = HALF
SPROWS = HALF + 8
E = 800000
EPS = E // 16
EPAD = 50176
EBUF = EPAD
D = 64
NLAYERS = 3
SENTINEL = 1 << 30

_MESH = dict(core_axis_name="c", subcore_axis_name="s", num_cores=2,
             num_subcores=16)

_f32 = jnp.float32
_i32 = jnp.int32


def _pad_shift(v):
    return v + jnp.where(v >= NU, _i32(88), _i32(0))


def _zero_rows(buf, nrows, width):
    z = jnp.zeros((16,), _f32)

    def body(i, _):
        for q in range(width // 16):
            buf[i, pl.ds(q * 16, 16)] = z
        return 0

    lax.fori_loop(0, nrows, body, 0)


def _deg_body(rows_hbm, deg_hbm, rows_v, ones_v, idx_v, deg_v, sem, deg_sp):
    c = lax.axis_index("c")
    s = lax.axis_index("s")
    tid = c * 16 + s
    lo = c * HALF

    _zero_rows(deg_v, 112, 16)

    def zcp(j, _):
        pltpu.sync_copy(deg_v, deg_sp.at[pl.ds(s * PT + j * 112, 112)])
        return 0

    lax.fori_loop(0, 14, zcp, 0)

    @pl.when(s == 0)
    def _():
        pltpu.sync_copy(deg_v.at[pl.ds(0, 8)], deg_sp.at[pl.ds(HALF, 8)])

    plsc.subcore_barrier()

    pltpu.sync_copy(rows_hbm.at[pl.ds(s * EPS, EPS)],
                    rows_v.at[pl.ds(0, EPS)])
    sent = jnp.full((16,), SENTINEL, _i32)
    for k in range(11):
        rows_v[pl.ds(EPS + k * 16, 16)] = sent

    one = jnp.ones((16,), _f32)

    def ones_body(i, _):
        ones_v[i, pl.ds(0, 16)] = one
        return 0

    lax.fori_loop(0, 128, ones_body, 0)

    def chunk(j, _):
        for u in range(8):
            r = rows_v[pl.ds(j * 128 + u * 16, 16)]
            pr = _pad_shift(r)
            m = (pr >= lo) & (pr < lo + HALF)
            loc = jnp.where(m, pr - lo, _i32(TRASH))
            idx_v[pl.ds(u * 16, 16)] = loc
        pltpu.sync_copy(ones_v, deg_sp.at[idx_v], add=True)
        return 0

    lax.fori_loop(0, EPAD // 128, chunk, 0)

    plsc.subcore_barrier()

    def wb(j, _):
        pltpu.sync_copy(deg_sp.at[pl.ds(s * PT + j * 112, 112)], deg_v)
        pltpu.sync_copy(deg_v,
                        deg_hbm.at[pl.ds(lo + s * PT + j * 112, 112)])
        return 0

    lax.fori_loop(0, 14, wb, 0)


def _deg_kernel(rows):
    mesh = plsc.VectorSubcoreMesh(**_MESH)
    return pl.kernel(
        _deg_body,
        out_type=jax.ShapeDtypeStruct((NP, 16), _f32),
        mesh=mesh,
        compiler_params=pltpu.CompilerParams(use_tc_tiling_on_sc=False),
        scratch_types=[
            pltpu.VMEM((EBUF,), _i32),
            pltpu.VMEM((128, 16), _f32),
            pltpu.VMEM((128,), _i32),
            pltpu.VMEM((112, 16), _f32),
            pltpu.SemaphoreType.DMA,
            pltpu.VMEM_SHARED((SPROWS, 16), _f32),
        ],
    )(rows)


def _scale_body(deg_ref, x0_ref, dinv_ref, dinv2_ref, y0_ref):
    d = deg_ref[:, 0:1]
    dv = jnp.where(d > 0.0, lax.rsqrt(jnp.maximum(d, 1e-30)), 0.0)
    dinv_ref[:, :] = dv
    dinv2_ref[:, :] = dv * dv
    y0_ref[:, :] = x0_ref[:, :] * dv


def _scale_kernel(deg16, x0p):
    grid = (NP // 128,)
    return pl.pallas_call(
        _scale_body,
        grid=grid,
        in_specs=[
            pl.BlockSpec((128, 16), lambda i: (i, 0)),
            pl.BlockSpec((128, D), lambda i: (i, 0)),
        ],
        out_specs=[
            pl.BlockSpec((128, 1), lambda i: (i, 0)),
            pl.BlockSpec((128, 1), lambda i: (i, 0)),
            pl.BlockSpec((128, D), lambda i: (i, 0)),
        ],
        out_shape=[
            jax.ShapeDtypeStruct((NP, 1), _f32),
            jax.ShapeDtypeStruct((NP, 1), _f32),
            jax.ShapeDtypeStruct((NP, D), _f32),
        ],
    )(deg16, x0p)


def _prop_body(rows_hbm, cols_hbm, y_hbm, dinv2_hbm, s_hbm, ynext_hbm,
               rows_v, cols_v, g0, idx_v, cidx_v, dinv2_v, sbuf, ybuf, sem,
               s_sp):
    c = lax.axis_index("c")
    s = lax.axis_index("s")
    tid = c * 16 + s
    lo = c * HALF

    _zero_rows(sbuf, 112, D)

    def zcp(j, _):
        pltpu.sync_copy(sbuf, s_sp.at[pl.ds(s * PT + j * 112, 112)])
        return 0

    lax.fori_loop(0, 14, zcp, 0)

    @pl.when(s == 0)
    def _():
        pltpu.sync_copy(sbuf.at[pl.ds(0, 8)], s_sp.at[pl.ds(HALF, 8)])

    plsc.subcore_barrier()

    sent = jnp.full((16,), SENTINEL, _i32)
    zero16 = jnp.zeros((16,), _i32)

    def gs(j, _):
        for u in range(8):
            r = rows_v[pl.ds(j * 128 + u * 16, 16)]
            cl = cols_v[pl.ds(j * 128 + u * 16, 16)]
            pr = _pad_shift(r)
            pc = _pad_shift(cl)
            m = (pr >= lo) & (pr < lo + HALF)
            idx_v[pl.ds(u * 16, 16)] = jnp.where(m, pr - lo, _i32(TRASH))
            cidx_v[pl.ds(u * 16, 16)] = pc
        pltpu.async_copy(y_hbm.at[cidx_v], g0, sem).wait()
        pltpu.sync_copy(g0, s_sp.at[idx_v], add=True)
        return 0

    for b in range(28):
        if b < 27:
            pltpu.sync_copy(rows_hbm.at[pl.ds(s * EPS + b * 1792, 1792)],
                            rows_v.at[pl.ds(0, 1792)])
            pltpu.sync_copy(cols_hbm.at[pl.ds(s * EPS + b * 1792, 1792)],
                            cols_v.at[pl.ds(0, 1792)])
        else:
            pltpu.sync_copy(rows_hbm.at[pl.ds(s * EPS + b * 1792, 1616)],
                            rows_v.at[pl.ds(0, 1616)])
            pltpu.sync_copy(cols_hbm.at[pl.ds(s * EPS + b * 1792, 1616)],
                            cols_v.at[pl.ds(0, 1616)])
            for k in range(11):
                rows_v[pl.ds(1616 + k * 16, 16)] = sent
                cols_v[pl.ds(1616 + k * 16, 16)] = zero16
        lax.fori_loop(0, 14, gs, 0)

    plsc.subcore_barrier()

    pltpu.sync_copy(dinv2_hbm.at[pl.ds(lo + s * PT, PT)],
                    dinv2_v.at[pl.ds(0, PT)])

    def wb(j, _):
        pltpu.sync_copy(s_sp.at[pl.ds(s * PT + j * 112, 112)], sbuf)
        pltpu.sync_copy(sbuf,
                        s_hbm.at[pl.ds(lo + s * PT + j * 112, 112)])

        def row(i, _):
            dv2 = dinv2_v[pl.ds(j * 112 + i, 16)][0]
            for q in range(D // 16):
                ybuf[i, pl.ds(q * 16, 16)] = sbuf[i, pl.ds(q * 16, 16)] * dv2
            return 0

        lax.fori_loop(0, 112, row, 0)
        pltpu.sync_copy(ybuf,
                        ynext_hbm.at[pl.ds(lo + s * PT + j * 112, 112)])
        return 0

    lax.fori_loop(0, 14, wb, 0)


def _prop_kernel(rows, cols, y, dinv2):
    mesh = plsc.VectorSubcoreMesh(**_MESH)
    return pl.kernel(
        _prop_body,
        out_type=(
            jax.ShapeDtypeStruct((NP, D), _f32),
            jax.ShapeDtypeStruct((NP, D), _f32),
        ),
        mesh=mesh,
        compiler_params=pltpu.CompilerParams(use_tc_tiling_on_sc=False),
        scratch_types=[
            pltpu.VMEM((1808,), _i32),
            pltpu.VMEM((1808,), _i32),
            pltpu.VMEM((128, D), _f32),
            pltpu.VMEM((128,), _i32),
            pltpu.VMEM((128,), _i32),
            pltpu.VMEM((PT + 16,), _f32),
            pltpu.VMEM((112, D), _f32),
            pltpu.VMEM((112, D), _f32),
            pltpu.SemaphoreType.DMA,
            pltpu.VMEM_SHARED((SPROWS, D), _f32),
        ],
    )(rows, cols, y, dinv2)


def _final_body(x0_ref, s0_ref, s1_ref, s2_ref, dinv_ref, out_ref):
    acc = s0_ref[:, :] + s1_ref[:, :] + s2_ref[:, :]
    out_ref[:, :] = 0.25 * (x0_ref[:, :] + dinv_ref[:, :] * acc)


def _final_kernel(x0p, s0, s1, s2, dinv):
    grid = (NP // 128,)
    bs = pl.BlockSpec((128, D), lambda i: (i, 0))
    return pl.pallas_call(
        _final_body,
        grid=grid,
        in_specs=[bs, bs, bs, bs, pl.BlockSpec((128, 1), lambda i: (i, 0))],
        out_specs=bs,
        out_shape=jax.ShapeDtypeStruct((NP, D), _f32),
    )(x0p, s0, s1, s2, dinv)


def kernel(edge_index, user_emb, item_emb):
    rows = edge_index[0]
    cols = edge_index[1]
    pad = jnp.zeros((HALF - NU, D), _f32)
    x0p = jnp.concatenate([user_emb, pad, item_emb, pad], axis=0)

    _DBG_JNP_DEG = False
    if _DBG_JNP_DEG:
        degj = jnp.zeros((2 * NU,), _f32).at[rows].add(1.0)
        degp = jnp.concatenate([degj[:NU], jnp.zeros((HALF - NU,), _f32),
                                degj[NU:], jnp.zeros((HALF - NU,), _f32)])
        deg16 = jnp.broadcast_to(degp[:, None], (NP, 16))
    else:
        deg16 = _deg_kernel(rows)
    dinv, dinv2, y = _scale_kernel(deg16, x0p)
    dinv2_flat = dinv2.reshape((NP,))

    ss = []
    for _ in range(NLAYERS):
        s_k, y = _prop_kernel(rows, cols, y, dinv2_flat)
        ss.append(s_k)

    finalp = _final_kernel(x0p, ss[0], ss[1], ss[2], dinv)
    users = finalp[:NU]
    items = finalp[HALF:HALF + NU]
    return (users, items)


# speedup vs baseline: 7.6018x; 1.0678x over previous
import functools

import jax
import jax.numpy as jnp
from jax import lax
from jax.experimental import pallas as pl
from jax.experimental.pallas import tpu as pltpu
from jax.experimental.pallas import tpu_sc as plsc

NU = 25000
NP = 50176
HALF = 25088
PT = 1568
TRASH = HALF
SPROWS = HALF + 8
E = 800000
EPS = E // 16
EPAD = 50176
EBUF = EPAD
D = 64
NLAYERS = 3
SENTINEL = 1 << 30

_MESH = dict(core_axis_name="c", subcore_axis_name="s", num_cores=2,
             num_subcores=16)

_f32 = jnp.float32
_i32 = jnp.int32


def _pad_shift(v):
    return v + jnp.where(v >= NU, _i32(88), _i32(0))


def _zero_rows(buf, nrows, width):
    z = jnp.zeros((16,), _f32)

    def body(i, _):
        for q in range(width // 16):
            buf[i, pl.ds(q * 16, 16)] = z
        return 0

    lax.fori_loop(0, nrows, body, 0)


def _deg_body(rows_hbm, deg_hbm, rows_v, ones_v, idx_v0, idx_v1, idx_v2,
              idx_v3, deg_v, sem0, sem1, sem2, sem3, deg_sp):
    sems = (sem0, sem1, sem2, sem3)
    c = lax.axis_index("c")
    s = lax.axis_index("s")
    tid = c * 16 + s
    lo = c * HALF

    _zero_rows(deg_v, 112, 16)

    def zcp(j, _):
        pltpu.sync_copy(deg_v, deg_sp.at[pl.ds(s * PT + j * 112, 112)])
        return 0

    lax.fori_loop(0, 14, zcp, 0)

    @pl.when(s == 0)
    def _():
        pltpu.sync_copy(deg_v.at[pl.ds(0, 8)], deg_sp.at[pl.ds(HALF, 8)])

    plsc.subcore_barrier()

    pltpu.sync_copy(rows_hbm.at[pl.ds(s * EPS, EPS)],
                    rows_v.at[pl.ds(0, EPS)])
    sent = jnp.full((16,), SENTINEL, _i32)
    for k in range(11):
        rows_v[pl.ds(EPS + k * 16, 16)] = sent

    one = jnp.ones((16,), _f32)

    def ones_body(i, _):
        ones_v[i, pl.ds(0, 16)] = one
        return 0

    lax.fori_loop(0, 128, ones_body, 0)

    idx_slots = (idx_v0, idx_v1, idx_v2, idx_v3)

    def group(i, _):
        descs = []
        for k in range(4):
            idx_k = idx_slots[k]
            for u in range(8):
                r = rows_v[pl.ds(i * 512 + k * 128 + u * 16, 16)]
                pr = _pad_shift(r)
                m = (pr >= lo) & (pr < lo + HALF)
                idx_k[pl.ds(u * 16, 16)] = jnp.where(m, pr - lo, _i32(TRASH))
            descs.append(pltpu.async_copy(ones_v, deg_sp.at[idx_k], sems[k],
                                          add=True))
        for d in descs:
            d.wait()
        return 0

    lax.fori_loop(0, EPAD // 512, group, 0)

    plsc.subcore_barrier()

    def wb(j, _):
        pltpu.sync_copy(deg_sp.at[pl.ds(s * PT + j * 112, 112)], deg_v)
        pltpu.sync_copy(deg_v,
                        deg_hbm.at[pl.ds(lo + s * PT + j * 112, 112)])
        return 0

    lax.fori_loop(0, 14, wb, 0)


def _deg_kernel(rows):
    mesh = plsc.VectorSubcoreMesh(**_MESH)
    return pl.kernel(
        _deg_body,
        out_type=jax.ShapeDtypeStruct((NP, 16), _f32),
        mesh=mesh,
        compiler_params=pltpu.CompilerParams(use_tc_tiling_on_sc=False),
        scratch_types=[
            pltpu.VMEM((EBUF,), _i32),
            pltpu.VMEM((128, 16), _f32),
            pltpu.VMEM((128,), _i32),
            pltpu.VMEM((128,), _i32),
            pltpu.VMEM((128,), _i32),
            pltpu.VMEM((128,), _i32),
            pltpu.VMEM((112, 16), _f32),
            pltpu.SemaphoreType.DMA,
            pltpu.SemaphoreType.DMA,
            pltpu.SemaphoreType.DMA,
            pltpu.SemaphoreType.DMA,
            pltpu.VMEM_SHARED((SPROWS, 16), _f32),
        ],
    )(rows)


def _scale_body(deg_ref, x0_ref, dinv_ref, dinv2_ref, y0_ref):
    d = deg_ref[:, 0:1]
    dv = jnp.where(d > 0.0, lax.rsqrt(jnp.maximum(d, 1e-30)), 0.0)
    dinv_ref[:, :] = dv
    dinv2_ref[:, :] = dv * dv
    y0_ref[:, :] = x0_ref[:, :] * dv


def _scale_kernel(deg16, x0p):
    grid = (NP // 128,)
    return pl.pallas_call(
        _scale_body,
        grid=grid,
        in_specs=[
            pl.BlockSpec((128, 16), lambda i: (i, 0)),
            pl.BlockSpec((128, D), lambda i: (i, 0)),
        ],
        out_specs=[
            pl.BlockSpec((128, 1), lambda i: (i, 0)),
            pl.BlockSpec((128, 1), lambda i: (i, 0)),
            pl.BlockSpec((128, D), lambda i: (i, 0)),
        ],
        out_shape=[
            jax.ShapeDtypeStruct((NP, 1), _f32),
            jax.ShapeDtypeStruct((NP, 1), _f32),
            jax.ShapeDtypeStruct((NP, D), _f32),
        ],
    )(deg16, x0p)


def _prop_body(rows_hbm, cols_hbm, y_hbm, dinv2_hbm, s_hbm, ynext_hbm,
               rows_v, cols_v, g0, g1, idx0_v, cidx0_v, idx1_v, cidx1_v,
               dinv2_v, sbuf, ybuf, semg0, semg1, sems0, sems1, s_sp):
    c = lax.axis_index("c")
    s = lax.axis_index("s")
    lo = c * HALF

    _zero_rows(sbuf, 56, D)

    def zcp(j, _):
        pltpu.sync_copy(sbuf, s_sp.at[pl.ds(s * PT + j * 56, 56)])
        return 0

    lax.fori_loop(0, 28, zcp, 0)

    @pl.when(s == 0)
    def _():
        pltpu.sync_copy(sbuf.at[pl.ds(0, 8)], s_sp.at[pl.ds(HALF, 8)])

    plsc.subcore_barrier()

    sent = jnp.full((16,), SENTINEL, _i32)
    zero16 = jnp.zeros((16,), _i32)

    def fill(base, idx_v, cidx_v):
        for u in range(8):
            r = rows_v[pl.ds(base + u * 16, 16)]
            cl = cols_v[pl.ds(base + u * 16, 16)]
            pr = _pad_shift(r)
            pc = _pad_shift(cl)
            m = (pr >= lo) & (pr < lo + HALF)
            idx_v[pl.ds(u * 16, 16)] = jnp.where(m, pr - lo, _i32(TRASH))
            cidx_v[pl.ds(u * 16, 16)] = pc

    def pair(i, _):
        fill(i * 256, idx0_v, cidx0_v)
        dg0 = pltpu.async_copy(y_hbm.at[cidx0_v], g0, semg0)
        fill(i * 256 + 128, idx1_v, cidx1_v)
        dg1 = pltpu.async_copy(y_hbm.at[cidx1_v], g1, semg1)
        dg0.wait()
        ds0 = pltpu.async_copy(g0, s_sp.at[idx0_v], sems0, add=True)
        dg1.wait()
        ds1 = pltpu.async_copy(g1, s_sp.at[idx1_v], sems1, add=True)
        ds0.wait()
        ds1.wait()
        return 0

    def block(b, _):
        pltpu.sync_copy(rows_hbm.at[pl.ds(s * EPS + b * 1792, 1792)],
                        rows_v.at[pl.ds(0, 1792)])
        pltpu.sync_copy(cols_hbm.at[pl.ds(s * EPS + b * 1792, 1792)],
                        cols_v.at[pl.ds(0, 1792)])
        lax.fori_loop(0, 7, pair, 0)
        return 0

    lax.fori_loop(0, 27, block, 0)

    pltpu.sync_copy(rows_hbm.at[pl.ds(s * EPS + 27 * 1792, 1616)],
                    rows_v.at[pl.ds(0, 1616)])
    pltpu.sync_copy(cols_hbm.at[pl.ds(s * EPS + 27 * 1792, 1616)],
                    cols_v.at[pl.ds(0, 1616)])
    for k in range(11):
        rows_v[pl.ds(1616 + k * 16, 16)] = sent
        cols_v[pl.ds(1616 + k * 16, 16)] = zero16
    lax.fori_loop(0, 7, pair, 0)

    plsc.subcore_barrier()

    pltpu.sync_copy(dinv2_hbm.at[pl.ds(lo + s * PT, PT)],
                    dinv2_v.at[pl.ds(0, PT)])

    def wb(j, _):
        pltpu.sync_copy(s_sp.at[pl.ds(s * PT + j * 56, 56)], sbuf)
        pltpu.sync_copy(sbuf,
                        s_hbm.at[pl.ds(lo + s * PT + j * 56, 56)])

        def row(i, _):
            dv2 = dinv2_v[pl.ds(j * 56 + i, 16)][0]
            for q in range(D // 16):
                ybuf[i, pl.ds(q * 16, 16)] = sbuf[i, pl.ds(q * 16, 16)] * dv2
            return 0

        lax.fori_loop(0, 56, row, 0)
        pltpu.sync_copy(ybuf,
                        ynext_hbm.at[pl.ds(lo + s * PT + j * 56, 56)])
        return 0

    lax.fori_loop(0, 28, wb, 0)


def _prop_kernel(rows, cols, y, dinv2):
    mesh = plsc.VectorSubcoreMesh(**_MESH)
    return pl.kernel(
        _prop_body,
        out_type=(
            jax.ShapeDtypeStruct((NP, D), _f32),
            jax.ShapeDtypeStruct((NP, D), _f32),
        ),
        mesh=mesh,
        compiler_params=pltpu.CompilerParams(use_tc_tiling_on_sc=False),
        scratch_types=[
            pltpu.VMEM((1808,), _i32),
            pltpu.VMEM((1808,), _i32),
            pltpu.VMEM((128, D), _f32),
            pltpu.VMEM((128, D), _f32),
            pltpu.VMEM((128,), _i32),
            pltpu.VMEM((128,), _i32),
            pltpu.VMEM((128,), _i32),
            pltpu.VMEM((128,), _i32),
            pltpu.VMEM((PT + 16,), _f32),
            pltpu.VMEM((56, D), _f32),
            pltpu.VMEM((56, D), _f32),
            pltpu.SemaphoreType.DMA,
            pltpu.SemaphoreType.DMA,
            pltpu.SemaphoreType.DMA,
            pltpu.SemaphoreType.DMA,
            pltpu.VMEM_SHARED((SPROWS, D), _f32),
        ],
    )(rows, cols, y, dinv2)


def _final_body(x0_ref, s0_ref, s1_ref, s2_ref, dinv_ref, out_ref):
    acc = s0_ref[:, :] + s1_ref[:, :] + s2_ref[:, :]
    out_ref[:, :] = 0.25 * (x0_ref[:, :] + dinv_ref[:, :] * acc)


def _final_kernel(x0p, s0, s1, s2, dinv):
    grid = (NP // 128,)
    bs = pl.BlockSpec((128, D), lambda i: (i, 0))
    return pl.pallas_call(
        _final_body,
        grid=grid,
        in_specs=[bs, bs, bs, bs, pl.BlockSpec((128, 1), lambda i: (i, 0))],
        out_specs=bs,
        out_shape=jax.ShapeDtypeStruct((NP, D), _f32),
    )(x0p, s0, s1, s2, dinv)


def kernel(edge_index, user_emb, item_emb):
    rows = edge_index[0]
    cols = edge_index[1]
    pad = jnp.zeros((HALF - NU, D), _f32)
    x0p = jnp.concatenate([user_emb, pad, item_emb, pad], axis=0)

    _DBG_JNP_DEG = False
    if _DBG_JNP_DEG:
        degj = jnp.zeros((2 * NU,), _f32).at[rows].add(1.0)
        degp = jnp.concatenate([degj[:NU], jnp.zeros((HALF - NU,), _f32),
                                degj[NU:], jnp.zeros((HALF - NU,), _f32)])
        deg16 = jnp.broadcast_to(degp[:, None], (NP, 16))
    else:
        deg16 = _deg_kernel(rows)
    dinv, dinv2, y = _scale_kernel(deg16, x0p)
    dinv2_flat = dinv2.reshape((NP,))

    ss = []
    for _ in range(NLAYERS):
        s_k, y = _prop_kernel(rows, cols, y, dinv2_flat)
        ss.append(s_k)

    finalp = _final_kernel(x0p, ss[0], ss[1], ss[2], dinv)
    users = finalp[:NU]
    items = finalp[HALF:HALF + NU]
    return (users, items)
